# baseline (speedup 1.0000x reference)
import functools

import jax
import jax.numpy as jnp
from jax import lax
from jax.experimental import pallas as pl
from jax.experimental.pallas import tpu as pltpu
from jax.experimental.pallas import tpu_sc as plsc

HID = 128
N_OBJ = 2048
N_REL = 16384
HALF = N_REL // 2
NUM_OBJ_CLS = 151
NUM_REL_CLS = 51

NC = 2
NS = 16
NW = NC * NS
EPW = HALF // NW
_CH = EPW // 2
_ROWS_PER_TILE = N_OBJ // NS

_DN_T = (((1,), (1,)), ((), ()))


def _sigmoid(x):
    return jax.nn.sigmoid(x)


def _gru_tail(gi, gh, h):
    r = _sigmoid(gi[:, :HID] + gh[:, :HID])
    z = _sigmoid(gi[:, HID:2 * HID] + gh[:, HID:2 * HID])
    n = jnp.tanh(gi[:, 2 * HID:] + r * gh[:, 2 * HID:])
    return (1.0 - z) * n + z * h


def _dotT(x, w):
    return lax.dot_general(x, w, _DN_T, preferred_element_type=jnp.float32)


def _dotT_bf(x, w):
    return lax.dot_general(x.astype(jnp.bfloat16), w.astype(jnp.bfloat16),
                           _DN_T, preferred_element_type=jnp.float32)


def _edge_prologue_body(uf_ref, we_ref, be_ref, wih_ref, bih_ref, bhh_ref,
                        out_ref):
    rel = jnp.maximum(_dotT_bf(uf_ref[...], we_ref[...]) + be_ref[...], 0.0)
    gi = _dotT_bf(rel, wih_ref[...]) + bih_ref[...]
    bhh = bhh_ref[...]
    r = _sigmoid(gi[:, :HID] + bhh[:, :HID])
    z = _sigmoid(gi[:, HID:2 * HID] + bhh[:, HID:2 * HID])
    n = jnp.tanh(gi[:, 2 * HID:] + r * bhh[:, 2 * HID:])
    out_ref[...] = (1.0 - z) * n


def _edge_prologue(uf, we, be, wih, bih, bhh, half):
    BE = 2048
    nblk = HALF // BE
    off = half * nblk
    return pl.pallas_call(
        _edge_prologue_body,
        grid=(nblk,),
        in_specs=[
            pl.BlockSpec((BE, uf.shape[1]), lambda i: (i + off, 0)),
            pl.BlockSpec(we.shape, lambda i: (0, 0)),
            pl.BlockSpec(be.shape, lambda i: (0, 0)),
            pl.BlockSpec(wih.shape, lambda i: (0, 0)),
            pl.BlockSpec(bih.shape, lambda i: (0, 0)),
            pl.BlockSpec(bhh.shape, lambda i: (0, 0)),
        ],
        out_specs=pl.BlockSpec((BE, HID), lambda i: (i, 0)),
        out_shape=jax.ShapeDtypeStruct((HALF, HID), jnp.float32),
    )(uf, we, be, wih, bih, bhh)


def _vert_prologue_body(x_ref, wo_ref, bo_ref, wih_ref, bih_ref, bhh_ref,
                        out_ref):
    obj_rep = _dotT(x_ref[...], wo_ref[...]) + bo_ref[...]
    gi = _dotT(obj_rep, wih_ref[...]) + bih_ref[...]
    bhh = bhh_ref[...]
    r = _sigmoid(gi[:, :HID] + bhh[:, :HID])
    z = _sigmoid(gi[:, HID:2 * HID] + bhh[:, HID:2 * HID])
    n = jnp.tanh(gi[:, 2 * HID:] + r * bhh[:, 2 * HID:])
    out_ref[...] = (1.0 - z) * n


def _vert_prologue(x, wo, bo, wih, bih, bhh):
    return pl.pallas_call(
        _vert_prologue_body,
        out_shape=jax.ShapeDtypeStruct((N_OBJ, HID), jnp.float32),
    )(x, wo, bo, wih, bih, bhh)


def _edge_gates(sub, obj, edge, gvT, geT, gb):
    su = jnp.dot(sub, gvT, preferred_element_type=jnp.float32)
    ob = jnp.dot(obj, gvT, preferred_element_type=jnp.float32)
    ed = jnp.dot(edge, geT, preferred_element_type=jnp.float32)
    w_s = _sigmoid(su[:, 0:1] + ed[:, 0:1] + gb[0:1, 0:1])
    w_o = _sigmoid(ob[:, 1:2] + ed[:, 1:2] + gb[0:1, 1:2])
    g_out = _sigmoid(su[:, 2:3] + ed[:, 2:3] + gb[0:1, 2:3])
    g_in = _sigmoid(ob[:, 3:4] + ed[:, 3:4] + gb[0:1, 3:4])
    return w_s, w_o, g_out, g_in


def _edge_iter_body(sub_ref, obj_ref, edge_ref, gvT_ref, geT_ref, gb_ref,
                    wih_ref, whh_ref, bih_ref, bhh_ref,
                    eo_ref, po_ref, pi_ref):
    sub = sub_ref[...]
    obj = obj_ref[...]
    edge = edge_ref[...]
    w_s, w_o, g_out, g_in = _edge_gates(sub, obj, edge, gvT_ref[...],
                                        geT_ref[...], gb_ref[...])
    x_in = w_s * sub + w_o * obj
    gi = _dotT_bf(x_in, wih_ref[...]) + bih_ref[...]
    gh = _dotT_bf(edge, whh_ref[...]) + bhh_ref[...]
    eo_ref[...] = _gru_tail(gi, gh, edge)
    po_ref[...] = g_out * edge
    pi_ref[...] = g_in * edge


def _edge_iter(sub_v, obj_v, edge, gvT, geT, gb, wih, whh, bih, bhh):
    BE = 1024
    grid = (HALF // BE,)
    row = lambda i: (i, 0)
    rep = lambda i: (0, 0)
    return pl.pallas_call(
        _edge_iter_body,
        grid=grid,
        in_specs=[
            pl.BlockSpec((BE, HID), row),
            pl.BlockSpec((BE, HID), row),
            pl.BlockSpec((BE, HID), row),
            pl.BlockSpec(gvT.shape, rep),
            pl.BlockSpec(geT.shape, rep),
            pl.BlockSpec(gb.shape, rep),
            pl.BlockSpec(wih.shape, rep),
            pl.BlockSpec(whh.shape, rep),
            pl.BlockSpec(bih.shape, rep),
            pl.BlockSpec(bhh.shape, rep),
        ],
        out_specs=[
            pl.BlockSpec((BE, HID), row),
            pl.BlockSpec((BE, HID), row),
            pl.BlockSpec((BE, HID), row),
        ],
        out_shape=[
            jax.ShapeDtypeStruct((HALF, HID), jnp.float32),
            jax.ShapeDtypeStruct((HALF, HID), jnp.float32),
            jax.ShapeDtypeStruct((HALF, HID), jnp.float32),
        ],
    )(sub_v, obj_v, edge, gvT, geT, gb, wih, whh, bih, bhh)


def _edge_iter_last_body(sub_ref, obj_ref, edge_ref, gvT_ref, geT_ref, gb_ref,
                         wih_ref, whh_ref, bih_ref, bhh_ref,
                         fc_ref, fcb_ref, rdp_ref, rd_ref, po_ref, pi_ref):
    del rdp_ref
    sub = sub_ref[...]
    obj = obj_ref[...]
    edge = edge_ref[...]
    w_s, w_o, g_out, g_in = _edge_gates(sub, obj, edge, gvT_ref[...],
                                        geT_ref[...], gb_ref[...])
    x_in = w_s * sub + w_o * obj
    gi = _dotT_bf(x_in, wih_ref[...]) + bih_ref[...]
    gh = _dotT_bf(edge, whh_ref[...]) + bhh_ref[...]
    edge_new = _gru_tail(gi, gh, edge)
    rd_ref[...] = _dotT(edge_new, fc_ref[...]) + fcb_ref[...]
    po_ref[...] = g_out * edge
    pi_ref[...] = g_in * edge


def _edge_iter_last(sub_v, obj_v, edge, gvT, geT, gb, wih, whh, bih, bhh,
                    fc, fcb, rd_prev, half):
    BE = 1024
    nblk = HALF // BE
    off = half * nblk
    row = lambda i: (i, 0)
    rowo = lambda i: (i + off, 0)
    rep = lambda i: (0, 0)
    ncls = fc.shape[0]
    return pl.pallas_call(
        _edge_iter_last_body,
        grid=(nblk,),
        in_specs=[
            pl.BlockSpec((BE, HID), row),
            pl.BlockSpec((BE, HID), row),
            pl.BlockSpec((BE, HID), row),
            pl.BlockSpec(gvT.shape, rep),
            pl.BlockSpec(geT.shape, rep),
            pl.BlockSpec(gb.shape, rep),
            pl.BlockSpec(wih.shape, rep),
            pl.BlockSpec(whh.shape, rep),
            pl.BlockSpec(bih.shape, rep),
            pl.BlockSpec(bhh.shape, rep),
            pl.BlockSpec(fc.shape, rep),
            pl.BlockSpec(fcb.shape, rep),
            pl.BlockSpec((BE, ncls), rowo),
        ],
        out_specs=[
            pl.BlockSpec((BE, ncls), rowo),
            pl.BlockSpec((BE, HID), row),
            pl.BlockSpec((BE, HID), row),
        ],
        out_shape=[
            jax.ShapeDtypeStruct((N_REL, ncls), jnp.float32),
            jax.ShapeDtypeStruct((HALF, HID), jnp.float32),
            jax.ShapeDtypeStruct((HALF, HID), jnp.float32),
        ],
        input_output_aliases={12: 0},
    )(sub_v, obj_v, edge, gvT, geT, gb, wih, whh, bih, bhh, fc, fcb, rd_prev)


def _node_iter_body(s0_ref, s1_ref, vert_ref, wih_ref, whh_ref, bih_ref,
                    bhh_ref, out_ref):
    ctx = (s0_ref[:N_OBJ, :] + s0_ref[N_OBJ:, :]
           + s1_ref[:N_OBJ, :] + s1_ref[N_OBJ:, :])
    vert = vert_ref[...]
    gi = _dotT(ctx, wih_ref[...]) + bih_ref[...]
    gh = _dotT(vert, whh_ref[...]) + bhh_ref[...]
    out_ref[...] = _gru_tail(gi, gh, vert)


def _node_iter(s0, s1, vert, wih, whh, bih, bhh):
    return pl.pallas_call(
        _node_iter_body,
        out_shape=jax.ShapeDtypeStruct((N_OBJ, HID), jnp.float32),
    )(s0, s1, vert, wih, whh, bih, bhh)


def _node_iter_last_body(s0_ref, s1_ref, vert_ref, wih_ref, whh_ref, bih_ref,
                         bhh_ref, fc_ref, fcb_ref, out_ref):
    ctx = (s0_ref[:N_OBJ, :] + s0_ref[N_OBJ:, :]
           + s1_ref[:N_OBJ, :] + s1_ref[N_OBJ:, :])
    vert = vert_ref[...]
    gi = _dotT(ctx, wih_ref[...]) + bih_ref[...]
    gh = _dotT(vert, whh_ref[...]) + bhh_ref[...]
    vert_new = _gru_tail(gi, gh, vert)
    out_ref[...] = _dotT(vert_new, fc_ref[...]) + fcb_ref[...]


def _node_iter_last(s0, s1, vert, wih, whh, bih, bhh, fc, fcb):
    return pl.pallas_call(
        _node_iter_last_body,
        out_shape=jax.ShapeDtypeStruct((N_OBJ, fc.shape[0]), jnp.float32),
    )(s0, s1, vert, wih, whh, bih, bhh, fc, fcb)


_SC_MESH = plsc.VectorSubcoreMesh(core_axis_name="c", subcore_axis_name="s")


def _make_gather(off):
    @functools.partial(
        pl.kernel,
        mesh=_SC_MESH,
        out_type=[
            jax.ShapeDtypeStruct((HALF, HID), jnp.float32),
            jax.ShapeDtypeStruct((HALF, HID), jnp.float32),
        ],
        scratch_types=[
            pltpu.VMEM((EPW,), jnp.int32),
            pltpu.VMEM((EPW,), jnp.int32),
            pltpu.VMEM((EPW, HID), jnp.float32),
            pltpu.VMEM((EPW, HID), jnp.float32),
            pltpu.SemaphoreType.DMA,
            pltpu.SemaphoreType.DMA,
            pltpu.SemaphoreType.DMA,
            pltpu.SemaphoreType.DMA,
        ],
    )
    def gather(table_hbm, sidx_hbm, oidx_hbm, sub_out, obj_out,
               idx_s, idx_o, buf_a, buf_b, sa, sb, wa, wb):
        wid = lax.axis_index("s") * NC + lax.axis_index("c")
        bo = wid * EPW
        bi = off + bo
        ia = pltpu.async_copy(sidx_hbm.at[pl.ds(bi, EPW)], idx_s, sa)
        ib = pltpu.async_copy(oidx_hbm.at[pl.ds(bi, EPW)], idx_o, sb)
        ia.wait()
        g0 = pltpu.async_copy(table_hbm.at[idx_s], buf_a, sa)
        ib.wait()
        g1 = pltpu.async_copy(table_hbm.at[idx_o], buf_b, sb)
        g0.wait()
        w0 = pltpu.async_copy(buf_a, sub_out.at[pl.ds(bo, EPW)], wa)
        g1.wait()
        w1 = pltpu.async_copy(buf_b, obj_out.at[pl.ds(bo, EPW)], wb)
        w0.wait()
        w1.wait()

    return gather


def _make_scatter(off):
    @functools.partial(
        pl.kernel,
        mesh=_SC_MESH,
        out_type=jax.ShapeDtypeStruct((NC * N_OBJ, HID), jnp.float32),
        scratch_types=[
            pltpu.VMEM((EPW,), jnp.int32),
            pltpu.VMEM((EPW,), jnp.int32),
            pltpu.VMEM((EPW, HID), jnp.float32),
            pltpu.VMEM((EPW, HID), jnp.float32),
            pltpu.VMEM_SHARED((N_OBJ, HID), jnp.float32),
            pltpu.SemaphoreType.DMA,
            pltpu.SemaphoreType.DMA,
            pltpu.SemaphoreType.DMA,
        ],
    )
    def scatter(po_hbm, pi_hbm, sidx_hbm, oidx_hbm, zeros_hbm, out_hbm,
                idx_s, idx_o, buf_a, buf_b, acc, rs_a, rs_b, zs):
        c = lax.axis_index("c")
        s = lax.axis_index("s")
        z = pltpu.async_copy(zeros_hbm, acc.at[pl.ds(s * _ROWS_PER_TILE,
                                                     _ROWS_PER_TILE)], zs)
        bo = (c * NS + s) * EPW
        bi = off + bo
        ia = pltpu.async_copy(sidx_hbm.at[pl.ds(bi, EPW)], idx_s, rs_a)
        ib = pltpu.async_copy(oidx_hbm.at[pl.ds(bi, EPW)], idx_o, rs_b)
        ia.wait()
        ib.wait()
        r0 = pltpu.async_copy(po_hbm.at[pl.ds(bo, EPW)], buf_a, rs_a)
        r1 = pltpu.async_copy(pi_hbm.at[pl.ds(bo, EPW)], buf_b, rs_b)
        z.wait()
        plsc.subcore_barrier()
        r0.wait()
        pltpu.sync_copy(buf_a, acc.at[idx_s], add=True)
        r1.wait()
        pltpu.sync_copy(buf_b, acc.at[idx_o], add=True)
        plsc.subcore_barrier()
        src = acc.at[pl.ds(s * _ROWS_PER_TILE, _ROWS_PER_TILE)]
        pltpu.sync_copy(src, out_hbm.at[pl.ds(c * N_OBJ + s * _ROWS_PER_TILE,
                                              _ROWS_PER_TILE)])

    return scatter


_gather_half = (_make_gather(0), _make_gather(HALF))
_scatter_half = (_make_scatter(0), _make_scatter(HALF))


def kernel(x, union_features, rel_pair_idxs, obj_unary_w, obj_unary_b,
           edge_unary_w, edge_unary_b, node_w_ih, node_w_hh, node_b_ih,
           node_b_hh, edge_w_ih, edge_w_hh, edge_b_ih, edge_b_hh, sub_w,
           sub_b, obj_w, obj_b, out_w, out_b, in_w, in_b, obj_fc_w,
           obj_fc_b, rel_fc_w, rel_fc_b):
    f32 = jnp.float32
    sidx = rel_pair_idxs[:, 0].astype(jnp.int32)
    oidx = rel_pair_idxs[:, 1].astype(jnp.int32)

    e_bih = edge_b_ih.reshape(1, -1)
    e_bhh = edge_b_hh.reshape(1, -1)
    n_bih = node_b_ih.reshape(1, -1)
    n_bhh = node_b_hh.reshape(1, -1)
    be = edge_unary_b.reshape(1, -1)
    bo = obj_unary_b.reshape(1, -1)
    obj_fcb = obj_fc_b.reshape(1, -1)
    rel_fcb = rel_fc_b.reshape(1, -1)

    gvT = jnp.zeros((HID, 128), f32).at[:, :4].set(
        jnp.stack([sub_w[0, :HID], obj_w[0, :HID],
                   out_w[0, :HID], in_w[0, :HID]], axis=1))
    geT = jnp.zeros((HID, 128), f32).at[:, :4].set(
        jnp.stack([sub_w[0, HID:], obj_w[0, HID:],
                   out_w[0, HID:], in_w[0, HID:]], axis=1))
    gb = jnp.zeros((1, 128), f32).at[0, :4].set(
        jnp.stack([sub_b[0], obj_b[0], out_b[0], in_b[0]]))

    vert = _vert_prologue(x, obj_unary_w, bo, node_w_ih, n_bih, n_bhh)
    e0 = _edge_prologue(union_features, edge_unary_w, be, edge_w_ih,
                        e_bih, e_bhh, 0)
    e1 = _edge_prologue(union_features, edge_unary_w, be, edge_w_ih,
                        e_bih, e_bhh, 1)
    edge = [e0, e1]

    zeros_tile = jnp.zeros((_ROWS_PER_TILE, HID), f32)

    for _ in range(2):
        scat = [None, None]
        new_edge = [None, None]
        for h in (0, 1):
            sub_v, obj_v = _gather_half[h](vert, sidx, oidx)
            new_edge[h], po, pi = _edge_iter(
                sub_v, obj_v, edge[h], gvT, geT, gb,
                edge_w_ih, edge_w_hh, e_bih, e_bhh)
            scat[h] = _scatter_half[h](po, pi, sidx, oidx, zeros_tile)
        vert = _node_iter(scat[0], scat[1], vert,
                          node_w_ih, node_w_hh, n_bih, n_bhh)
        edge = new_edge

    scat = [None, None]
    rel_dists = jnp.zeros((N_REL, NUM_REL_CLS), f32)
    for h in (0, 1):
        sub_v, obj_v = _gather_half[h](vert, sidx, oidx)
        rel_dists, po, pi = _edge_iter_last(
            sub_v, obj_v, edge[h], gvT, geT, gb,
            edge_w_ih, edge_w_hh, e_bih, e_bhh, rel_fc_w, rel_fcb,
            rel_dists, h)
        scat[h] = _scatter_half[h](po, pi, sidx, oidx, zeros_tile)
    obj_dists = _node_iter_last(scat[0], scat[1], vert, node_w_ih,
                                node_w_hh, n_bih, n_bhh, obj_fc_w, obj_fcb)
    return (obj_dists, rel_dists)


# speedup vs baseline: 1.0141x; 1.0141x over previous
import functools

import jax
import jax.numpy as jnp
from jax import lax
from jax.experimental import pallas as pl
from jax.experimental.pallas import tpu as pltpu
from jax.experimental.pallas import tpu_sc as plsc

HID = 128
N_OBJ = 2048
N_REL = 16384
HALF = N_REL // 2
NUM_OBJ_CLS = 151
NUM_REL_CLS = 51

NC = 2
NS = 16
NW = NC * NS
EPW = HALF // NW
_CH = EPW // 2
_ROWS_PER_TILE = N_OBJ // NS

_DN_T = (((1,), (1,)), ((), ()))


def _sigmoid(x):
    return jax.nn.sigmoid(x)


def _gru_tail(gi, gh, h):
    r = _sigmoid(gi[:, :HID] + gh[:, :HID])
    z = _sigmoid(gi[:, HID:2 * HID] + gh[:, HID:2 * HID])
    n = jnp.tanh(gi[:, 2 * HID:] + r * gh[:, 2 * HID:])
    return (1.0 - z) * n + z * h


def _dotT(x, w):
    return lax.dot_general(x, w, _DN_T, preferred_element_type=jnp.float32)


def _dotT_bf(x, w):
    return lax.dot_general(x.astype(jnp.bfloat16), w.astype(jnp.bfloat16),
                           _DN_T, preferred_element_type=jnp.float32)


def _edge_prologue_body(uf_ref, we_ref, be_ref, wih_ref, bih_ref, bhh_ref,
                        out_ref):
    rel = jnp.maximum(_dotT_bf(uf_ref[...], we_ref[...]) + be_ref[...], 0.0)
    gi = _dotT_bf(rel, wih_ref[...]) + bih_ref[...]
    bhh = bhh_ref[...]
    r = _sigmoid(gi[:, :HID] + bhh[:, :HID])
    z = _sigmoid(gi[:, HID:2 * HID] + bhh[:, HID:2 * HID])
    n = jnp.tanh(gi[:, 2 * HID:] + r * bhh[:, 2 * HID:])
    out_ref[...] = (1.0 - z) * n


def _edge_prologue(uf, we, be, wih, bih, bhh, half):
    BE = 1024
    nblk = HALF // BE
    off = half * nblk
    return pl.pallas_call(
        _edge_prologue_body,
        grid=(nblk,),
        in_specs=[
            pl.BlockSpec((BE, uf.shape[1]), lambda i: (i + off, 0)),
            pl.BlockSpec(we.shape, lambda i: (0, 0)),
            pl.BlockSpec(be.shape, lambda i: (0, 0)),
            pl.BlockSpec(wih.shape, lambda i: (0, 0)),
            pl.BlockSpec(bih.shape, lambda i: (0, 0)),
            pl.BlockSpec(bhh.shape, lambda i: (0, 0)),
        ],
        out_specs=pl.BlockSpec((BE, HID), lambda i: (i, 0)),
        out_shape=jax.ShapeDtypeStruct((HALF, HID), jnp.float32),
    )(uf, we, be, wih, bih, bhh)


def _vert_prologue_body(x_ref, wo_ref, bo_ref, wih_ref, bih_ref, bhh_ref,
                        out_ref):
    obj_rep = _dotT(x_ref[...], wo_ref[...]) + bo_ref[...]
    gi = _dotT(obj_rep, wih_ref[...]) + bih_ref[...]
    bhh = bhh_ref[...]
    r = _sigmoid(gi[:, :HID] + bhh[:, :HID])
    z = _sigmoid(gi[:, HID:2 * HID] + bhh[:, HID:2 * HID])
    n = jnp.tanh(gi[:, 2 * HID:] + r * bhh[:, 2 * HID:])
    out_ref[...] = (1.0 - z) * n


def _vert_prologue(x, wo, bo, wih, bih, bhh):
    return pl.pallas_call(
        _vert_prologue_body,
        out_shape=jax.ShapeDtypeStruct((N_OBJ, HID), jnp.float32),
    )(x, wo, bo, wih, bih, bhh)


def _edge_gates(sub, obj, edge, gvT, geT, gb):
    su = jnp.dot(sub, gvT, preferred_element_type=jnp.float32)
    ob = jnp.dot(obj, gvT, preferred_element_type=jnp.float32)
    ed = jnp.dot(edge, geT, preferred_element_type=jnp.float32)
    w_s = _sigmoid(su[:, 0:1] + ed[:, 0:1] + gb[0:1, 0:1])
    w_o = _sigmoid(ob[:, 1:2] + ed[:, 1:2] + gb[0:1, 1:2])
    g_out = _sigmoid(su[:, 2:3] + ed[:, 2:3] + gb[0:1, 2:3])
    g_in = _sigmoid(ob[:, 3:4] + ed[:, 3:4] + gb[0:1, 3:4])
    return w_s, w_o, g_out, g_in


def _edge_iter_body(sub_ref, obj_ref, edge_ref, gvT_ref, geT_ref, gb_ref,
                    wih_ref, whh_ref, bih_ref, bhh_ref,
                    eo_ref, po_ref, pi_ref):
    sub = sub_ref[...]
    obj = obj_ref[...]
    edge = edge_ref[...]
    w_s, w_o, g_out, g_in = _edge_gates(sub, obj, edge, gvT_ref[...],
                                        geT_ref[...], gb_ref[...])
    x_in = w_s * sub + w_o * obj
    gi = _dotT_bf(x_in, wih_ref[...]) + bih_ref[...]
    gh = _dotT_bf(edge, whh_ref[...]) + bhh_ref[...]
    eo_ref[...] = _gru_tail(gi, gh, edge)
    po_ref[...] = g_out * edge
    pi_ref[...] = g_in * edge


def _edge_iter(sub_v, obj_v, edge, gvT, geT, gb, wih, whh, bih, bhh):
    BE = 2048
    grid = (HALF // BE,)
    row = lambda i: (i, 0)
    rep = lambda i: (0, 0)
    return pl.pallas_call(
        _edge_iter_body,
        grid=grid,
        in_specs=[
            pl.BlockSpec((BE, HID), row),
            pl.BlockSpec((BE, HID), row),
            pl.BlockSpec((BE, HID), row),
            pl.BlockSpec(gvT.shape, rep),
            pl.BlockSpec(geT.shape, rep),
            pl.BlockSpec(gb.shape, rep),
            pl.BlockSpec(wih.shape, rep),
            pl.BlockSpec(whh.shape, rep),
            pl.BlockSpec(bih.shape, rep),
            pl.BlockSpec(bhh.shape, rep),
        ],
        out_specs=[
            pl.BlockSpec((BE, HID), row),
            pl.BlockSpec((BE, HID), row),
            pl.BlockSpec((BE, HID), row),
        ],
        out_shape=[
            jax.ShapeDtypeStruct((HALF, HID), jnp.float32),
            jax.ShapeDtypeStruct((HALF, HID), jnp.float32),
            jax.ShapeDtypeStruct((HALF, HID), jnp.float32),
        ],
    )(sub_v, obj_v, edge, gvT, geT, gb, wih, whh, bih, bhh)


def _edge_iter_last_body(sub_ref, obj_ref, edge_ref, gvT_ref, geT_ref, gb_ref,
                         wih_ref, whh_ref, bih_ref, bhh_ref,
                         fc_ref, fcb_ref, rdp_ref, rd_ref, po_ref, pi_ref):
    del rdp_ref
    sub = sub_ref[...]
    obj = obj_ref[...]
    edge = edge_ref[...]
    w_s, w_o, g_out, g_in = _edge_gates(sub, obj, edge, gvT_ref[...],
                                        geT_ref[...], gb_ref[...])
    x_in = w_s * sub + w_o * obj
    gi = _dotT_bf(x_in, wih_ref[...]) + bih_ref[...]
    gh = _dotT_bf(edge, whh_ref[...]) + bhh_ref[...]
    edge_new = _gru_tail(gi, gh, edge)
    rd_ref[...] = _dotT(edge_new, fc_ref[...]) + fcb_ref[...]
    po_ref[...] = g_out * edge
    pi_ref[...] = g_in * edge


def _edge_iter_last(sub_v, obj_v, edge, gvT, geT, gb, wih, whh, bih, bhh,
                    fc, fcb, rd_prev, half):
    BE = 2048
    nblk = HALF // BE
    off = half * nblk
    row = lambda i: (i, 0)
    rowo = lambda i: (i + off, 0)
    rep = lambda i: (0, 0)
    ncls = fc.shape[0]
    return pl.pallas_call(
        _edge_iter_last_body,
        grid=(nblk,),
        in_specs=[
            pl.BlockSpec((BE, HID), row),
            pl.BlockSpec((BE, HID), row),
            pl.BlockSpec((BE, HID), row),
            pl.BlockSpec(gvT.shape, rep),
            pl.BlockSpec(geT.shape, rep),
            pl.BlockSpec(gb.shape, rep),
            pl.BlockSpec(wih.shape, rep),
            pl.BlockSpec(whh.shape, rep),
            pl.BlockSpec(bih.shape, rep),
            pl.BlockSpec(bhh.shape, rep),
            pl.BlockSpec(fc.shape, rep),
            pl.BlockSpec(fcb.shape, rep),
            pl.BlockSpec((BE, ncls), rowo),
        ],
        out_specs=[
            pl.BlockSpec((BE, ncls), rowo),
            pl.BlockSpec((BE, HID), row),
            pl.BlockSpec((BE, HID), row),
        ],
        out_shape=[
            jax.ShapeDtypeStruct((N_REL, ncls), jnp.float32),
            jax.ShapeDtypeStruct((HALF, HID), jnp.float32),
            jax.ShapeDtypeStruct((HALF, HID), jnp.float32),
        ],
        input_output_aliases={12: 0},
    )(sub_v, obj_v, edge, gvT, geT, gb, wih, whh, bih, bhh, fc, fcb, rd_prev)


def _node_iter_body(s0_ref, s1_ref, vert_ref, wih_ref, whh_ref, bih_ref,
                    bhh_ref, out_ref):
    ctx = (s0_ref[:N_OBJ, :] + s0_ref[N_OBJ:, :]
           + s1_ref[:N_OBJ, :] + s1_ref[N_OBJ:, :])
    vert = vert_ref[...]
    gi = _dotT(ctx, wih_ref[...]) + bih_ref[...]
    gh = _dotT(vert, whh_ref[...]) + bhh_ref[...]
    out_ref[...] = _gru_tail(gi, gh, vert)


def _node_iter(s0, s1, vert, wih, whh, bih, bhh):
    return pl.pallas_call(
        _node_iter_body,
        out_shape=jax.ShapeDtypeStruct((N_OBJ, HID), jnp.float32),
    )(s0, s1, vert, wih, whh, bih, bhh)


def _node_iter_last_body(s0_ref, s1_ref, vert_ref, wih_ref, whh_ref, bih_ref,
                         bhh_ref, fc_ref, fcb_ref, out_ref):
    ctx = (s0_ref[:N_OBJ, :] + s0_ref[N_OBJ:, :]
           + s1_ref[:N_OBJ, :] + s1_ref[N_OBJ:, :])
    vert = vert_ref[...]
    gi = _dotT(ctx, wih_ref[...]) + bih_ref[...]
    gh = _dotT(vert, whh_ref[...]) + bhh_ref[...]
    vert_new = _gru_tail(gi, gh, vert)
    out_ref[...] = _dotT(vert_new, fc_ref[...]) + fcb_ref[...]


def _node_iter_last(s0, s1, vert, wih, whh, bih, bhh, fc, fcb):
    return pl.pallas_call(
        _node_iter_last_body,
        out_shape=jax.ShapeDtypeStruct((N_OBJ, fc.shape[0]), jnp.float32),
    )(s0, s1, vert, wih, whh, bih, bhh, fc, fcb)


_SC_MESH = plsc.VectorSubcoreMesh(core_axis_name="c", subcore_axis_name="s")


def _make_gather(off):
    @functools.partial(
        pl.kernel,
        mesh=_SC_MESH,
        out_type=[
            jax.ShapeDtypeStruct((HALF, HID), jnp.float32),
            jax.ShapeDtypeStruct((HALF, HID), jnp.float32),
        ],
        scratch_types=[
            pltpu.VMEM((EPW,), jnp.int32),
            pltpu.VMEM((EPW,), jnp.int32),
            pltpu.VMEM((EPW, HID), jnp.float32),
            pltpu.VMEM((EPW, HID), jnp.float32),
            pltpu.SemaphoreType.DMA,
            pltpu.SemaphoreType.DMA,
            pltpu.SemaphoreType.DMA,
            pltpu.SemaphoreType.DMA,
        ],
    )
    def gather(table_hbm, sidx_hbm, oidx_hbm, sub_out, obj_out,
               idx_s, idx_o, buf_a, buf_b, sa, sb, wa, wb):
        wid = lax.axis_index("s") * NC + lax.axis_index("c")
        bo = wid * EPW
        bi = off + bo
        ia = pltpu.async_copy(sidx_hbm.at[pl.ds(bi, EPW)], idx_s, sa)
        ib = pltpu.async_copy(oidx_hbm.at[pl.ds(bi, EPW)], idx_o, sb)
        ia.wait()
        g0 = pltpu.async_copy(table_hbm.at[idx_s], buf_a, sa)
        ib.wait()
        g1 = pltpu.async_copy(table_hbm.at[idx_o], buf_b, sb)
        g0.wait()
        w0 = pltpu.async_copy(buf_a, sub_out.at[pl.ds(bo, EPW)], wa)
        g1.wait()
        w1 = pltpu.async_copy(buf_b, obj_out.at[pl.ds(bo, EPW)], wb)
        w0.wait()
        w1.wait()

    return gather


def _make_scatter(off):
    @functools.partial(
        pl.kernel,
        mesh=_SC_MESH,
        out_type=jax.ShapeDtypeStruct((NC * N_OBJ, HID), jnp.float32),
        scratch_types=[
            pltpu.VMEM((EPW,), jnp.int32),
            pltpu.VMEM((EPW,), jnp.int32),
            pltpu.VMEM((EPW, HID), jnp.float32),
            pltpu.VMEM((EPW, HID), jnp.float32),
            pltpu.VMEM_SHARED((N_OBJ, HID), jnp.float32),
            pltpu.SemaphoreType.DMA,
            pltpu.SemaphoreType.DMA,
            pltpu.SemaphoreType.DMA,
        ],
    )
    def scatter(po_hbm, pi_hbm, sidx_hbm, oidx_hbm, zeros_hbm, out_hbm,
                idx_s, idx_o, buf_a, buf_b, acc, rs_a, rs_b, zs):
        c = lax.axis_index("c")
        s = lax.axis_index("s")
        z = pltpu.async_copy(zeros_hbm, acc.at[pl.ds(s * _ROWS_PER_TILE,
                                                     _ROWS_PER_TILE)], zs)
        bo = (c * NS + s) * EPW
        bi = off + bo
        ia = pltpu.async_copy(sidx_hbm.at[pl.ds(bi, EPW)], idx_s, rs_a)
        ib = pltpu.async_copy(oidx_hbm.at[pl.ds(bi, EPW)], idx_o, rs_b)
        ia.wait()
        ib.wait()
        r0 = pltpu.async_copy(po_hbm.at[pl.ds(bo, EPW)], buf_a, rs_a)
        r1 = pltpu.async_copy(pi_hbm.at[pl.ds(bo, EPW)], buf_b, rs_b)
        z.wait()
        plsc.subcore_barrier()
        r0.wait()
        pltpu.sync_copy(buf_a, acc.at[idx_s], add=True)
        r1.wait()
        pltpu.sync_copy(buf_b, acc.at[idx_o], add=True)
        plsc.subcore_barrier()
        src = acc.at[pl.ds(s * _ROWS_PER_TILE, _ROWS_PER_TILE)]
        pltpu.sync_copy(src, out_hbm.at[pl.ds(c * N_OBJ + s * _ROWS_PER_TILE,
                                              _ROWS_PER_TILE)])

    return scatter


_gather_half = (_make_gather(0), _make_gather(HALF))
_scatter_half = (_make_scatter(0), _make_scatter(HALF))


def kernel(x, union_features, rel_pair_idxs, obj_unary_w, obj_unary_b,
           edge_unary_w, edge_unary_b, node_w_ih, node_w_hh, node_b_ih,
           node_b_hh, edge_w_ih, edge_w_hh, edge_b_ih, edge_b_hh, sub_w,
           sub_b, obj_w, obj_b, out_w, out_b, in_w, in_b, obj_fc_w,
           obj_fc_b, rel_fc_w, rel_fc_b):
    f32 = jnp.float32
    sidx = rel_pair_idxs[:, 0].astype(jnp.int32)
    oidx = rel_pair_idxs[:, 1].astype(jnp.int32)

    e_bih = edge_b_ih.reshape(1, -1)
    e_bhh = edge_b_hh.reshape(1, -1)
    n_bih = node_b_ih.reshape(1, -1)
    n_bhh = node_b_hh.reshape(1, -1)
    be = edge_unary_b.reshape(1, -1)
    bo = obj_unary_b.reshape(1, -1)
    obj_fcb = obj_fc_b.reshape(1, -1)
    rel_fcb = rel_fc_b.reshape(1, -1)

    gvT = jnp.zeros((HID, 128), f32).at[:, :4].set(
        jnp.stack([sub_w[0, :HID], obj_w[0, :HID],
                   out_w[0, :HID], in_w[0, :HID]], axis=1))
    geT = jnp.zeros((HID, 128), f32).at[:, :4].set(
        jnp.stack([sub_w[0, HID:], obj_w[0, HID:],
                   out_w[0, HID:], in_w[0, HID:]], axis=1))
    gb = jnp.zeros((1, 128), f32).at[0, :4].set(
        jnp.stack([sub_b[0], obj_b[0], out_b[0], in_b[0]]))

    vert = _vert_prologue(x, obj_unary_w, bo, node_w_ih, n_bih, n_bhh)
    e0 = _edge_prologue(union_features, edge_unary_w, be, edge_w_ih,
                        e_bih, e_bhh, 0)
    e1 = _edge_prologue(union_features, edge_unary_w, be, edge_w_ih,
                        e_bih, e_bhh, 1)
    edge = [e0, e1]

    zeros_tile = jnp.zeros((_ROWS_PER_TILE, HID), f32)

    for _ in range(2):
        scat = [None, None]
        new_edge = [None, None]
        for h in (0, 1):
            sub_v, obj_v = _gather_half[h](vert, sidx, oidx)
            new_edge[h], po, pi = _edge_iter(
                sub_v, obj_v, edge[h], gvT, geT, gb,
                edge_w_ih, edge_w_hh, e_bih, e_bhh)
            scat[h] = _scatter_half[h](po, pi, sidx, oidx, zeros_tile)
        vert = _node_iter(scat[0], scat[1], vert,
                          node_w_ih, node_w_hh, n_bih, n_bhh)
        edge = new_edge

    scat = [None, None]
    rel_dists = jnp.zeros((N_REL, NUM_REL_CLS), f32)
    for h in (0, 1):
        sub_v, obj_v = _gather_half[h](vert, sidx, oidx)
        rel_dists, po, pi = _edge_iter_last(
            sub_v, obj_v, edge[h], gvT, geT, gb,
            edge_w_ih, edge_w_hh, e_bih, e_bhh, rel_fc_w, rel_fcb,
            rel_dists, h)
        scat[h] = _scatter_half[h](po, pi, sidx, oidx, zeros_tile)
    obj_dists = _node_iter_last(scat[0], scat[1], vert, node_w_ih,
                                node_w_hh, n_bih, n_bhh, obj_fc_w, obj_fcb)
    return (obj_dists, rel_dists)


# speedup vs baseline: 1.0325x; 1.0181x over previous
import functools

import jax
import jax.numpy as jnp
from jax import lax
from jax.experimental import pallas as pl
from jax.experimental.pallas import tpu as pltpu
from jax.experimental.pallas import tpu_sc as plsc

HID = 128
N_OBJ = 2048
N_REL = 16384
HALF = N_REL // 2
NUM_OBJ_CLS = 151
NUM_REL_CLS = 51

NC = 2
NS = 16
NW = NC * NS
EPW = HALF // NW
_CH = EPW // 2
_ROWS_PER_TILE = N_OBJ // NS

_DN_T = (((1,), (1,)), ((), ()))


def _sigmoid(x):
    return jax.nn.sigmoid(x)


def _gru_tail(gi, gh, h):
    r = _sigmoid(gi[:, :HID] + gh[:, :HID])
    z = _sigmoid(gi[:, HID:2 * HID] + gh[:, HID:2 * HID])
    n = jnp.tanh(gi[:, 2 * HID:] + r * gh[:, 2 * HID:])
    return (1.0 - z) * n + z * h


def _dotT(x, w):
    return lax.dot_general(x, w, _DN_T, preferred_element_type=jnp.float32)


def _dotT_bf(x, w):
    return lax.dot_general(x.astype(jnp.bfloat16), w.astype(jnp.bfloat16),
                           _DN_T, preferred_element_type=jnp.float32)


def _edge_prologue_body(uf_ref, we_ref, be_ref, wih_ref, bih_ref, bhh_ref,
                        out_ref):
    rel = jnp.maximum(_dotT_bf(uf_ref[...], we_ref[...]) + be_ref[...], 0.0)
    gi = _dotT_bf(rel, wih_ref[...]) + bih_ref[...]
    bhh = bhh_ref[...]
    r = _sigmoid(gi[:, :HID] + bhh[:, :HID])
    z = _sigmoid(gi[:, HID:2 * HID] + bhh[:, HID:2 * HID])
    n = jnp.tanh(gi[:, 2 * HID:] + r * bhh[:, 2 * HID:])
    out_ref[...] = (1.0 - z) * n


def _edge_prologue(uf, we, be, wih, bih, bhh, half):
    BE = 1024
    nblk = HALF // BE
    off = half * nblk
    return pl.pallas_call(
        _edge_prologue_body,
        grid=(nblk,),
        in_specs=[
            pl.BlockSpec((BE, uf.shape[1]), lambda i: (i + off, 0)),
            pl.BlockSpec(we.shape, lambda i: (0, 0)),
            pl.BlockSpec(be.shape, lambda i: (0, 0)),
            pl.BlockSpec(wih.shape, lambda i: (0, 0)),
            pl.BlockSpec(bih.shape, lambda i: (0, 0)),
            pl.BlockSpec(bhh.shape, lambda i: (0, 0)),
        ],
        out_specs=pl.BlockSpec((BE, HID), lambda i: (i, 0)),
        out_shape=jax.ShapeDtypeStruct((HALF, HID), jnp.float32),
    )(uf, we, be, wih, bih, bhh)


def _vert_prologue_body(x_ref, wo_ref, bo_ref, wih_ref, bih_ref, bhh_ref,
                        out_ref):
    obj_rep = _dotT(x_ref[...], wo_ref[...]) + bo_ref[...]
    gi = _dotT(obj_rep, wih_ref[...]) + bih_ref[...]
    bhh = bhh_ref[...]
    r = _sigmoid(gi[:, :HID] + bhh[:, :HID])
    z = _sigmoid(gi[:, HID:2 * HID] + bhh[:, HID:2 * HID])
    n = jnp.tanh(gi[:, 2 * HID:] + r * bhh[:, 2 * HID:])
    out_ref[...] = (1.0 - z) * n


def _vert_prologue(x, wo, bo, wih, bih, bhh):
    return pl.pallas_call(
        _vert_prologue_body,
        out_shape=jax.ShapeDtypeStruct((N_OBJ, HID), jnp.float32),
    )(x, wo, bo, wih, bih, bhh)


def _edge_gates(sub, obj, edge, gvT, geT, gb):
    su = jnp.dot(sub, gvT, preferred_element_type=jnp.float32)
    ob = jnp.dot(obj, gvT, preferred_element_type=jnp.float32)
    ed = jnp.dot(edge, geT, preferred_element_type=jnp.float32)
    w_s = _sigmoid(su[:, 0:1] + ed[:, 0:1] + gb[0:1, 0:1])
    w_o = _sigmoid(ob[:, 1:2] + ed[:, 1:2] + gb[0:1, 1:2])
    g_out = _sigmoid(su[:, 2:3] + ed[:, 2:3] + gb[0:1, 2:3])
    g_in = _sigmoid(ob[:, 3:4] + ed[:, 3:4] + gb[0:1, 3:4])
    return w_s, w_o, g_out, g_in


def _edge_iter_body(sub_ref, obj_ref, edge_ref, gvT_ref, geT_ref, gb_ref,
                    wih_ref, whh_ref, bih_ref, bhh_ref,
                    eo_ref, po_ref, pi_ref):
    sub = sub_ref[...]
    obj = obj_ref[...]
    edge = edge_ref[...]
    w_s, w_o, g_out, g_in = _edge_gates(sub, obj, edge, gvT_ref[...],
                                        geT_ref[...], gb_ref[...])
    x_in = w_s * sub + w_o * obj
    gi = _dotT_bf(x_in, wih_ref[...]) + bih_ref[...]
    gh = _dotT_bf(edge, whh_ref[...]) + bhh_ref[...]
    eo_ref[...] = _gru_tail(gi, gh, edge)
    po_ref[...] = g_out * edge
    pi_ref[...] = g_in * edge


def _edge_iter(sub_v, obj_v, edge, gvT, geT, gb, wih, whh, bih, bhh):
    BE = 2048
    grid = (HALF // BE,)
    row = lambda i: (i, 0)
    rep = lambda i: (0, 0)
    return pl.pallas_call(
        _edge_iter_body,
        grid=grid,
        in_specs=[
            pl.BlockSpec((BE, HID), row),
            pl.BlockSpec((BE, HID), row),
            pl.BlockSpec((BE, HID), row),
            pl.BlockSpec(gvT.shape, rep),
            pl.BlockSpec(geT.shape, rep),
            pl.BlockSpec(gb.shape, rep),
            pl.BlockSpec(wih.shape, rep),
            pl.BlockSpec(whh.shape, rep),
            pl.BlockSpec(bih.shape, rep),
            pl.BlockSpec(bhh.shape, rep),
        ],
        out_specs=[
            pl.BlockSpec((BE, HID), row),
            pl.BlockSpec((BE, HID), row),
            pl.BlockSpec((BE, HID), row),
        ],
        out_shape=[
            jax.ShapeDtypeStruct((HALF, HID), jnp.float32),
            jax.ShapeDtypeStruct((HALF, HID), jnp.float32),
            jax.ShapeDtypeStruct((HALF, HID), jnp.float32),
        ],
    )(sub_v, obj_v, edge, gvT, geT, gb, wih, whh, bih, bhh)


def _edge_iter_last_body(sub_ref, obj_ref, edge_ref, gvT_ref, geT_ref, gb_ref,
                         wih_ref, whh_ref, bih_ref, bhh_ref,
                         fc_ref, fcb_ref, rd_ref, po_ref, pi_ref):
    sub = sub_ref[...]
    obj = obj_ref[...]
    edge = edge_ref[...]
    w_s, w_o, g_out, g_in = _edge_gates(sub, obj, edge, gvT_ref[...],
                                        geT_ref[...], gb_ref[...])
    x_in = w_s * sub + w_o * obj
    gi = _dotT_bf(x_in, wih_ref[...]) + bih_ref[...]
    gh = _dotT_bf(edge, whh_ref[...]) + bhh_ref[...]
    edge_new = _gru_tail(gi, gh, edge)
    rd_ref[...] = _dotT(edge_new, fc_ref[...]) + fcb_ref[...]
    po_ref[...] = g_out * edge
    pi_ref[...] = g_in * edge


def _edge_iter_last(sub_v, obj_v, edge, gvT, geT, gb, wih, whh, bih, bhh,
                    fc, fcb):
    BE = 2048
    grid = (HALF // BE,)
    row = lambda i: (i, 0)
    rep = lambda i: (0, 0)
    ncls = fc.shape[0]
    return pl.pallas_call(
        _edge_iter_last_body,
        grid=grid,
        in_specs=[
            pl.BlockSpec((BE, HID), row),
            pl.BlockSpec((BE, HID), row),
            pl.BlockSpec((BE, HID), row),
            pl.BlockSpec(gvT.shape, rep),
            pl.BlockSpec(geT.shape, rep),
            pl.BlockSpec(gb.shape, rep),
            pl.BlockSpec(wih.shape, rep),
            pl.BlockSpec(whh.shape, rep),
            pl.BlockSpec(bih.shape, rep),
            pl.BlockSpec(bhh.shape, rep),
            pl.BlockSpec(fc.shape, rep),
            pl.BlockSpec(fcb.shape, rep),
        ],
        out_specs=[
            pl.BlockSpec((BE, ncls), row),
            pl.BlockSpec((BE, HID), row),
            pl.BlockSpec((BE, HID), row),
        ],
        out_shape=[
            jax.ShapeDtypeStruct((HALF, ncls), jnp.float32),
            jax.ShapeDtypeStruct((HALF, HID), jnp.float32),
            jax.ShapeDtypeStruct((HALF, HID), jnp.float32),
        ],
    )(sub_v, obj_v, edge, gvT, geT, gb, wih, whh, bih, bhh, fc, fcb)


def _node_iter_body(s0_ref, s1_ref, vert_ref, wih_ref, whh_ref, bih_ref,
                    bhh_ref, out_ref):
    ctx = (s0_ref[:N_OBJ, :] + s0_ref[N_OBJ:, :]
           + s1_ref[:N_OBJ, :] + s1_ref[N_OBJ:, :])
    vert = vert_ref[...]
    gi = _dotT(ctx, wih_ref[...]) + bih_ref[...]
    gh = _dotT(vert, whh_ref[...]) + bhh_ref[...]
    out_ref[...] = _gru_tail(gi, gh, vert)


def _node_iter(s0, s1, vert, wih, whh, bih, bhh):
    return pl.pallas_call(
        _node_iter_body,
        out_shape=jax.ShapeDtypeStruct((N_OBJ, HID), jnp.float32),
    )(s0, s1, vert, wih, whh, bih, bhh)


def _node_iter_last_body(s0_ref, s1_ref, vert_ref, wih_ref, whh_ref, bih_ref,
                         bhh_ref, fc_ref, fcb_ref, out_ref):
    ctx = (s0_ref[:N_OBJ, :] + s0_ref[N_OBJ:, :]
           + s1_ref[:N_OBJ, :] + s1_ref[N_OBJ:, :])
    vert = vert_ref[...]
    gi = _dotT(ctx, wih_ref[...]) + bih_ref[...]
    gh = _dotT(vert, whh_ref[...]) + bhh_ref[...]
    vert_new = _gru_tail(gi, gh, vert)
    out_ref[...] = _dotT(vert_new, fc_ref[...]) + fcb_ref[...]


def _node_iter_last(s0, s1, vert, wih, whh, bih, bhh, fc, fcb):
    return pl.pallas_call(
        _node_iter_last_body,
        out_shape=jax.ShapeDtypeStruct((N_OBJ, fc.shape[0]), jnp.float32),
    )(s0, s1, vert, wih, whh, bih, bhh, fc, fcb)


_SC_MESH = plsc.VectorSubcoreMesh(core_axis_name="c", subcore_axis_name="s")


def _make_gather(off):
    @functools.partial(
        pl.kernel,
        mesh=_SC_MESH,
        out_type=[
            jax.ShapeDtypeStruct((HALF, HID), jnp.float32),
            jax.ShapeDtypeStruct((HALF, HID), jnp.float32),
        ],
        scratch_types=[
            pltpu.VMEM((EPW,), jnp.int32),
            pltpu.VMEM((EPW,), jnp.int32),
            pltpu.VMEM((EPW, HID), jnp.float32),
            pltpu.VMEM((EPW, HID), jnp.float32),
            pltpu.SemaphoreType.DMA,
            pltpu.SemaphoreType.DMA,
            pltpu.SemaphoreType.DMA,
            pltpu.SemaphoreType.DMA,
        ],
    )
    def gather(table_hbm, sidx_hbm, oidx_hbm, sub_out, obj_out,
               idx_s, idx_o, buf_a, buf_b, sa, sb, wa, wb):
        wid = lax.axis_index("s") * NC + lax.axis_index("c")
        bo = wid * EPW
        bi = off + bo
        ia = pltpu.async_copy(sidx_hbm.at[pl.ds(bi, EPW)], idx_s, sa)
        ib = pltpu.async_copy(oidx_hbm.at[pl.ds(bi, EPW)], idx_o, sb)
        ia.wait()
        g0 = pltpu.async_copy(table_hbm.at[idx_s], buf_a, sa)
        ib.wait()
        g1 = pltpu.async_copy(table_hbm.at[idx_o], buf_b, sb)
        g0.wait()
        w0 = pltpu.async_copy(buf_a, sub_out.at[pl.ds(bo, EPW)], wa)
        g1.wait()
        w1 = pltpu.async_copy(buf_b, obj_out.at[pl.ds(bo, EPW)], wb)
        w0.wait()
        w1.wait()

    return gather


def _make_scatter(off):
    @functools.partial(
        pl.kernel,
        mesh=_SC_MESH,
        out_type=jax.ShapeDtypeStruct((NC * N_OBJ, HID), jnp.float32),
        scratch_types=[
            pltpu.VMEM((EPW,), jnp.int32),
            pltpu.VMEM((EPW,), jnp.int32),
            pltpu.VMEM((EPW, HID), jnp.float32),
            pltpu.VMEM((EPW, HID), jnp.float32),
            pltpu.VMEM_SHARED((N_OBJ, HID), jnp.float32),
            pltpu.SemaphoreType.DMA,
            pltpu.SemaphoreType.DMA,
            pltpu.SemaphoreType.DMA,
        ],
    )
    def scatter(po_hbm, pi_hbm, sidx_hbm, oidx_hbm, zeros_hbm, out_hbm,
                idx_s, idx_o, buf_a, buf_b, acc, rs_a, rs_b, zs):
        c = lax.axis_index("c")
        s = lax.axis_index("s")
        z = pltpu.async_copy(zeros_hbm, acc.at[pl.ds(s * _ROWS_PER_TILE,
                                                     _ROWS_PER_TILE)], zs)
        bo = (c * NS + s) * EPW
        bi = off + bo
        ia = pltpu.async_copy(sidx_hbm.at[pl.ds(bi, EPW)], idx_s, rs_a)
        ib = pltpu.async_copy(oidx_hbm.at[pl.ds(bi, EPW)], idx_o, rs_b)
        ia.wait()
        ib.wait()
        r0 = pltpu.async_copy(po_hbm.at[pl.ds(bo, EPW)], buf_a, rs_a)
        r1 = pltpu.async_copy(pi_hbm.at[pl.ds(bo, EPW)], buf_b, rs_b)
        z.wait()
        plsc.subcore_barrier()
        r0.wait()
        pltpu.sync_copy(buf_a, acc.at[idx_s], add=True)
        r1.wait()
        pltpu.sync_copy(buf_b, acc.at[idx_o], add=True)
        plsc.subcore_barrier()
        src = acc.at[pl.ds(s * _ROWS_PER_TILE, _ROWS_PER_TILE)]
        pltpu.sync_copy(src, out_hbm.at[pl.ds(c * N_OBJ + s * _ROWS_PER_TILE,
                                              _ROWS_PER_TILE)])

    return scatter


_gather_half = (_make_gather(0), _make_gather(HALF))
_scatter_half = (_make_scatter(0), _make_scatter(HALF))


def kernel(x, union_features, rel_pair_idxs, obj_unary_w, obj_unary_b,
           edge_unary_w, edge_unary_b, node_w_ih, node_w_hh, node_b_ih,
           node_b_hh, edge_w_ih, edge_w_hh, edge_b_ih, edge_b_hh, sub_w,
           sub_b, obj_w, obj_b, out_w, out_b, in_w, in_b, obj_fc_w,
           obj_fc_b, rel_fc_w, rel_fc_b):
    f32 = jnp.float32
    sidx = rel_pair_idxs[:, 0].astype(jnp.int32)
    oidx = rel_pair_idxs[:, 1].astype(jnp.int32)

    e_bih = edge_b_ih.reshape(1, -1)
    e_bhh = edge_b_hh.reshape(1, -1)
    n_bih = node_b_ih.reshape(1, -1)
    n_bhh = node_b_hh.reshape(1, -1)
    be = edge_unary_b.reshape(1, -1)
    bo = obj_unary_b.reshape(1, -1)
    obj_fcb = obj_fc_b.reshape(1, -1)
    rel_fcb = rel_fc_b.reshape(1, -1)

    gvT = jnp.zeros((HID, 128), f32).at[:, :4].set(
        jnp.stack([sub_w[0, :HID], obj_w[0, :HID],
                   out_w[0, :HID], in_w[0, :HID]], axis=1))
    geT = jnp.zeros((HID, 128), f32).at[:, :4].set(
        jnp.stack([sub_w[0, HID:], obj_w[0, HID:],
                   out_w[0, HID:], in_w[0, HID:]], axis=1))
    gb = jnp.zeros((1, 128), f32).at[0, :4].set(
        jnp.stack([sub_b[0], obj_b[0], out_b[0], in_b[0]]))

    vert = _vert_prologue(x, obj_unary_w, bo, node_w_ih, n_bih, n_bhh)
    e0 = _edge_prologue(union_features, edge_unary_w, be, edge_w_ih,
                        e_bih, e_bhh, 0)
    e1 = _edge_prologue(union_features, edge_unary_w, be, edge_w_ih,
                        e_bih, e_bhh, 1)
    edge = [e0, e1]

    zeros_tile = jnp.zeros((_ROWS_PER_TILE, HID), f32)

    for _ in range(2):
        scat = [None, None]
        new_edge = [None, None]
        for h in (0, 1):
            sub_v, obj_v = _gather_half[h](vert, sidx, oidx)
            new_edge[h], po, pi = _edge_iter(
                sub_v, obj_v, edge[h], gvT, geT, gb,
                edge_w_ih, edge_w_hh, e_bih, e_bhh)
            scat[h] = _scatter_half[h](po, pi, sidx, oidx, zeros_tile)
        vert = _node_iter(scat[0], scat[1], vert,
                          node_w_ih, node_w_hh, n_bih, n_bhh)
        edge = new_edge

    scat = [None, None]
    rel = [None, None]
    for h in (0, 1):
        sub_v, obj_v = _gather_half[h](vert, sidx, oidx)
        rel[h], po, pi = _edge_iter_last(
            sub_v, obj_v, edge[h], gvT, geT, gb,
            edge_w_ih, edge_w_hh, e_bih, e_bhh, rel_fc_w, rel_fcb)
        scat[h] = _scatter_half[h](po, pi, sidx, oidx, zeros_tile)
    obj_dists = _node_iter_last(scat[0], scat[1], vert, node_w_ih,
                                node_w_hh, n_bih, n_bhh, obj_fc_w, obj_fcb)
    rel_dists = jnp.concatenate(rel, axis=0)
    return (obj_dists, rel_dists)
